# Initial kernel scaffold; baseline (speedup 1.0000x reference)
import jax
import jax.numpy as jnp
from jax.experimental import pallas as pl


def kernel(values_0, offsets_0, values_1, offsets_1, values_2, offsets_2, values_3, offsets_3, W_0, W_1, W_2, W_3):
    raise NotImplementedError("write your pallas kernel here")


# speedup vs baseline: 144.7098x; 144.7098x over previous
import functools

import jax
import jax.numpy as jnp
from jax import lax
from jax.experimental import pallas as pl
from jax.experimental.pallas import tpu as pltpu
from jax.experimental.pallas import tpu_sc as plsc

B = 16384
V = 100000
D = 64
T = 4
TOT = B * 20

NC = 2
NS = 16
NW = NC * NS
BPW = B // NW
C = 128

OFF_SLICE = 1040
OFF_PAD = (NW - 1) * BPW + OFF_SLICE
VAL_PAD = TOT + C
ACC_ROWS = BPW + 1


def _pool_body(vp0, op0, w0, vp1, op1, w1, vp2, op2, w2, vp3, op3, w3, zrows,
               out, off_v, vals_v, seg_v, rows_v, acc_sh):
  cid = lax.axis_index("c")
  sid = lax.axis_index("s")
  wid = sid * NC + cid
  bag0 = wid * BPW
  lane = lax.iota(jnp.int32, 16)
  acc0 = sid * ACC_ROWS

  for t, (vp, op, w) in enumerate(
      ((vp0, op0, w0), (vp1, op1, w1), (vp2, op2, w2), (vp3, op3, w3))):
    pltpu.sync_copy(zrows, acc_sh.at[pl.ds(acc0, ACC_ROWS)])
    pltpu.sync_copy(op.at[pl.ds(bag0, OFF_SLICE)], off_v)
    start = off_v[pl.ds(0, 16)][0]
    end = off_v[pl.ds(BPW, 16)][0]
    astart = lax.bitwise_and(start, jnp.int32(-8))
    nch = (end - astart + (C - 1)) // C

    def chunk(k, carry):
      base = pl.multiple_of(astart + k * C, 8)
      pltpu.sync_copy(vp.at[pl.ds(base, C)], vals_v)
      pltpu.sync_copy(w.at[vals_v], rows_v)
      for j16 in range(C // 16):
        pos = base + j16 * 16 + lane
        j = jnp.zeros((16,), jnp.int32)
        for sh in (512, 256, 128, 64, 32, 16, 8, 4, 2, 1):
          cand = j + sh
          ok = (cand <= BPW + 1) & (plsc.load_gather(off_v, [cand - 1]) <= pos)
          j = jnp.where(ok, cand, j)
        seg = jnp.where(j == 0, jnp.int32(BPW), j - 1) + acc0
        seg_v[pl.ds(j16 * 16, 16)] = seg
      pltpu.sync_copy(rows_v, acc_sh.at[seg_v], add=True)
      return carry

    lax.fori_loop(0, nch, chunk, 0)
    pltpu.sync_copy(acc_sh.at[pl.ds(acc0, BPW)], out.at[t, pl.ds(bag0, BPW)])


_pooled = pl.kernel(
    _pool_body,
    out_type=jax.ShapeDtypeStruct((T, B, D), jnp.float32),
    mesh=plsc.VectorSubcoreMesh(core_axis_name="c", subcore_axis_name="s"),
    compiler_params=pltpu.CompilerParams(
        needs_layout_passes=False, use_tc_tiling_on_sc=False),
    scratch_types=[
        pltpu.VMEM((OFF_SLICE,), jnp.int32),
        pltpu.VMEM((C,), jnp.int32),
        pltpu.VMEM((C,), jnp.int32),
        pltpu.VMEM((C, D), jnp.float32),
        pltpu.VMEM_SHARED((NS * ACC_ROWS, D), jnp.float32),
    ],
)


@jax.jit
def kernel(values_0, offsets_0, values_1, offsets_1, values_2, offsets_2,
           values_3, offsets_3, W_0, W_1, W_2, W_3):
  args = []
  for vals, offs, w in ((values_0, offsets_0, W_0), (values_1, offsets_1, W_1),
                        (values_2, offsets_2, W_2), (values_3, offsets_3, W_3)):
    vp = jnp.concatenate([vals, jnp.zeros((VAL_PAD - TOT,), jnp.int32)])
    op = jnp.concatenate(
        [offs, jnp.full((OFF_PAD - (B + 1),), TOT, jnp.int32)])
    args += [vp, op, w]
  zrows = jnp.zeros((ACC_ROWS, D), jnp.float32)
  pooled = _pooled(*args, zrows)
  return jnp.swapaxes(pooled, 0, 1).reshape(B, T * D)


# speedup vs baseline: 150.0861x; 1.0372x over previous
import functools

import jax
import jax.numpy as jnp
from jax import lax
from jax.experimental import pallas as pl
from jax.experimental.pallas import tpu as pltpu
from jax.experimental.pallas import tpu_sc as plsc

B = 16384
V = 100000
D = 64
T = 4
TOT = B * 20

NC = 2
NS = 16
NW = NC * NS
BPW = B // NW
C = 128

OFF_SLICE = 1040
OFF_PAD = (NW - 1) * BPW + OFF_SLICE
ACC_ROWS = BPW + 1


def _pool_body(vp0, op0, w0, vp1, op1, w1, vp2, op2, w2, vp3, op3, w3, zrows,
               out, off_v, vals_v, seg_v, rows_v, acc_sh):
  cid = lax.axis_index("c")
  sid = lax.axis_index("s")
  wid = sid * NC + cid
  bag0 = wid * BPW
  lane = lax.iota(jnp.int32, 16)
  acc0 = sid * ACC_ROWS

  for t, (vp, op, w) in enumerate(
      ((vp0, op0, w0), (vp1, op1, w1), (vp2, op2, w2), (vp3, op3, w3))):
    pltpu.sync_copy(zrows, acc_sh.at[pl.ds(acc0, ACC_ROWS)])
    pltpu.sync_copy(op.at[pl.ds(bag0, OFF_SLICE)], off_v)
    start = off_v[pl.ds(0, 16)][0]
    end = off_v[pl.ds(BPW, 16)][0]
    astart = lax.bitwise_and(start, jnp.int32(-8))
    nch = (end - astart + (C - 1)) // C

    def chunk(k, carry):
      base = astart + k * C
      cbase = pl.multiple_of(jnp.minimum(base, TOT - C), 8)
      pltpu.sync_copy(vp.at[pl.ds(cbase, C)], vals_v)
      pltpu.sync_copy(w.at[vals_v], rows_v)
      for j16 in range(C // 16):
        pos = cbase + j16 * 16 + lane
        j = jnp.zeros((16,), jnp.int32)
        for sh in (512, 256, 128, 64, 32, 16, 8, 4, 2, 1):
          cand = j + sh
          ok = (cand <= BPW + 1) & (plsc.load_gather(off_v, [cand - 1]) <= pos)
          j = jnp.where(ok, cand, j)
        seg = jnp.where((j == 0) | (pos < base), jnp.int32(BPW), j - 1) + acc0
        seg_v[pl.ds(j16 * 16, 16)] = seg
      pltpu.sync_copy(rows_v, acc_sh.at[seg_v], add=True)
      return carry

    lax.fori_loop(0, nch, chunk, 0)
    pltpu.sync_copy(acc_sh.at[pl.ds(acc0, BPW)],
                    out.at[pl.ds(bag0, BPW), pl.ds(t * D, D)])


_pooled = pl.kernel(
    _pool_body,
    out_type=jax.ShapeDtypeStruct((B, T * D), jnp.float32),
    mesh=plsc.VectorSubcoreMesh(core_axis_name="c", subcore_axis_name="s"),
    compiler_params=pltpu.CompilerParams(
        needs_layout_passes=False, use_tc_tiling_on_sc=False),
    scratch_types=[
        pltpu.VMEM((OFF_SLICE,), jnp.int32),
        pltpu.VMEM((C,), jnp.int32),
        pltpu.VMEM((C,), jnp.int32),
        pltpu.VMEM((C, D), jnp.float32),
        pltpu.VMEM_SHARED((NS * ACC_ROWS, D), jnp.float32),
    ],
)


@jax.jit
def kernel(values_0, offsets_0, values_1, offsets_1, values_2, offsets_2,
           values_3, offsets_3, W_0, W_1, W_2, W_3):
  args = []
  for vals, offs, w in ((values_0, offsets_0, W_0), (values_1, offsets_1, W_1),
                        (values_2, offsets_2, W_2), (values_3, offsets_3, W_3)):
    op = jnp.concatenate(
        [offs, jnp.full((OFF_PAD - (B + 1),), TOT, jnp.int32)])
    args += [vals, op, w]
  zrows = jnp.zeros((ACC_ROWS, D), jnp.float32)
  return _pooled(*args, zrows)


# speedup vs baseline: 150.1238x; 1.0003x over previous
import functools

import jax
import jax.numpy as jnp
from jax import lax
from jax.experimental import pallas as pl
from jax.experimental.pallas import tpu as pltpu
from jax.experimental.pallas import tpu_sc as plsc

B = 16384
V = 100000
D = 64
T = 4
TOT = B * 20

NC = 2
NS = 16
NW = NC * NS
BPW = B // NW
C = 128

OFF_SLICE = 1040
ACC_ROWS = BPW + 1


def _pool_body(vp0, op0, w0, vp1, op1, w1, vp2, op2, w2, vp3, op3, w3, zrows,
               out, off_v, vals_v, seg_v, rows_v, acc_sh):
  cid = lax.axis_index("c")
  sid = lax.axis_index("s")
  wid = sid * NC + cid
  bag0 = wid * BPW
  lane = lax.iota(jnp.int32, 16)
  acc0 = sid * ACC_ROWS

  for t, (vp, op, w) in enumerate(
      ((vp0, op0, w0), (vp1, op1, w1), (vp2, op2, w2), (vp3, op3, w3))):
    pltpu.sync_copy(zrows, acc_sh.at[pl.ds(acc0, ACC_ROWS)])
    pltpu.sync_copy(op.at[pl.ds(bag0, BPW + 1)], off_v.at[pl.ds(0, BPW + 1)])
    start = off_v[pl.ds(0, 16)][0]
    end = off_v[pl.ds(BPW, 16)][0]
    astart = lax.bitwise_and(start, jnp.int32(-8))
    nch = (end - astart + (C - 1)) // C

    def chunk(k, carry):
      base = astart + k * C
      cbase = pl.multiple_of(jnp.minimum(base, TOT - C), 8)
      pltpu.sync_copy(vp.at[pl.ds(cbase, C)], vals_v)
      pltpu.sync_copy(w.at[vals_v], rows_v)
      for j16 in range(C // 16):
        pos = cbase + j16 * 16 + lane
        j = jnp.zeros((16,), jnp.int32)
        for sh in (512, 256, 128, 64, 32, 16, 8, 4, 2, 1):
          cand = j + sh
          ok = (cand <= BPW + 1) & (plsc.load_gather(off_v, [cand - 1]) <= pos)
          j = jnp.where(ok, cand, j)
        seg = jnp.where((j == 0) | (pos < base), jnp.int32(BPW), j - 1) + acc0
        seg_v[pl.ds(j16 * 16, 16)] = seg
      pltpu.sync_copy(rows_v, acc_sh.at[seg_v], add=True)
      return carry

    lax.fori_loop(0, nch, chunk, 0)
    pltpu.sync_copy(acc_sh.at[pl.ds(acc0, BPW)],
                    out.at[pl.ds(bag0, BPW), pl.ds(t * D, D)])


_pooled = pl.kernel(
    _pool_body,
    out_type=jax.ShapeDtypeStruct((B, T * D), jnp.float32),
    mesh=plsc.VectorSubcoreMesh(core_axis_name="c", subcore_axis_name="s"),
    compiler_params=pltpu.CompilerParams(
        needs_layout_passes=False, use_tc_tiling_on_sc=False),
    scratch_types=[
        pltpu.VMEM((OFF_SLICE,), jnp.int32),
        pltpu.VMEM((C,), jnp.int32),
        pltpu.VMEM((C,), jnp.int32),
        pltpu.VMEM((C, D), jnp.float32),
        pltpu.VMEM_SHARED((NS * ACC_ROWS, D), jnp.float32),
    ],
)


@jax.jit
def kernel(values_0, offsets_0, values_1, offsets_1, values_2, offsets_2,
           values_3, offsets_3, W_0, W_1, W_2, W_3):
  zrows = jnp.zeros((ACC_ROWS, D), jnp.float32)
  return _pooled(values_0, offsets_0, W_0, values_1, offsets_1, W_1,
                 values_2, offsets_2, W_2, values_3, offsets_3, W_3, zrows)


# speedup vs baseline: 320.3286x; 2.1338x over previous
import jax
import jax.numpy as jnp
from jax import lax
from jax.experimental import pallas as pl
from jax.experimental.pallas import tpu as pltpu
from jax.experimental.pallas import tpu_sc as plsc

B = 16384
V = 100000
D = 64
T = 4
TOT = B * 20

NC = 2
NS = 16
NW = NC * NS
BPW = B // NW
C = 128
NBUF = 4

OFF_SLICE = 1040
ACC_ROWS = BPW + 1


def _pool_body(vp0, op0, w0, vp1, op1, w1, vp2, op2, w2, vp3, op3, w3, zrows,
               out, off_v, vals4, seg4, rows4, acc_sh, sem_v, sem_g, sem_s):
  cid = lax.axis_index("c")
  sid = lax.axis_index("s")
  wid = sid * NC + cid
  bag0 = wid * BPW
  lane = lax.iota(jnp.int32, 16)
  acc0 = sid * ACC_ROWS

  for t, (vp, op, w) in enumerate(
      ((vp0, op0, w0), (vp1, op1, w1), (vp2, op2, w2), (vp3, op3, w3))):
    pltpu.sync_copy(zrows, acc_sh.at[pl.ds(acc0, ACC_ROWS)])
    pltpu.sync_copy(op.at[pl.ds(bag0, BPW + 1)], off_v.at[pl.ds(0, BPW + 1)])
    start = off_v[pl.ds(0, 16)][0]
    end = off_v[pl.ds(BPW, 16)][0]
    astart = lax.bitwise_and(start, jnp.int32(-8))
    nch = (end - astart + (C - 1)) // C

    def cbase_of(m):
      return pl.multiple_of(jnp.minimum(astart + m * C, TOT - C), 8)

    def vals_issue(m, b):
      pltpu.async_copy(vp.at[pl.ds(cbase_of(m), C)], vals4.at[b], sem_v)

    def vals_wait():
      pltpu.make_async_copy(vp.at[pl.ds(0, C)], vals4.at[0], sem_v).wait()

    def gather_issue(b):
      pltpu.async_copy(w.at[vals4.at[b]], rows4.at[b], sem_g)

    def gather_wait():
      pltpu.make_async_copy(w.at[vals4.at[0]], rows4.at[0], sem_g).wait()

    def scatter_issue(b):
      pltpu.async_copy(rows4.at[b], acc_sh.at[seg4.at[b]], sem_s, add=True)

    def scatter_wait():
      pltpu.make_async_copy(rows4.at[0], acc_sh.at[seg4.at[0]], sem_s).wait()

    def seg_compute(m, b):
      base = astart + m * C
      cbase = cbase_of(m)

      def one_vreg(j16, carry):
        pos = cbase + j16 * 16 + lane
        j = jnp.zeros((16,), jnp.int32)
        for sh in (512, 256, 128, 64, 32, 16, 8, 4, 2, 1):
          cand = j + sh
          ok = (cand <= BPW + 1) & (plsc.load_gather(off_v, [cand - 1]) <= pos)
          j = jnp.where(ok, cand, j)
        seg = jnp.where((j == 0) | (pos < base), jnp.int32(BPW), j - 1) + acc0
        seg4[b, pl.ds(pl.multiple_of(j16 * 16, 16), 16)] = seg
        return carry

      lax.fori_loop(0, C // 16, one_vreg, 0)

    for m in range(NBUF):
      pl.when(m < nch)(lambda m=m: vals_issue(m, m))
    for m in range(2):
      def _prime(m=m):
        vals_wait()
        gather_issue(m)
      pl.when(m < nch)(_prime)

    def group(g, carry):
      for jj in range(NBUF):
        k = g * NBUF + jj
        b2 = (jj + 2) % NBUF
        pl.when((k >= 2) & (k - 2 < nch))(scatter_wait)

        def _launch_next(k=k, b2=b2):
          vals_wait()
          gather_issue(b2)
        pl.when(k + 2 < nch)(_launch_next)

        def _consume(k=k, jj=jj):
          gather_wait()
          seg_compute(k, jj)
          scatter_issue(jj)
        pl.when(k < nch)(_consume)
        pl.when(k + NBUF < nch)(lambda k=k, jj=jj: vals_issue(k + NBUF, jj))
      return carry

    lax.fori_loop(0, (nch + NBUF - 1) // NBUF + 1, group, 0)
    pltpu.sync_copy(acc_sh.at[pl.ds(acc0, BPW)],
                    out.at[pl.ds(bag0, BPW), pl.ds(t * D, D)])


_pooled = pl.kernel(
    _pool_body,
    out_type=jax.ShapeDtypeStruct((B, T * D), jnp.float32),
    mesh=plsc.VectorSubcoreMesh(core_axis_name="c", subcore_axis_name="s"),
    compiler_params=pltpu.CompilerParams(
        needs_layout_passes=False, use_tc_tiling_on_sc=False),
    scratch_types=[
        pltpu.VMEM((OFF_SLICE,), jnp.int32),
        pltpu.VMEM((NBUF, C), jnp.int32),
        pltpu.VMEM((NBUF, C), jnp.int32),
        pltpu.VMEM((NBUF, C, D), jnp.float32),
        pltpu.VMEM_SHARED((NS * ACC_ROWS, D), jnp.float32),
        pltpu.SemaphoreType.DMA,
        pltpu.SemaphoreType.DMA,
        pltpu.SemaphoreType.DMA,
    ],
)


@jax.jit
def kernel(values_0, offsets_0, values_1, offsets_1, values_2, offsets_2,
           values_3, offsets_3, W_0, W_1, W_2, W_3):
  zrows = jnp.zeros((ACC_ROWS, D), jnp.float32)
  return _pooled(values_0, offsets_0, W_0, values_1, offsets_1, W_1,
                 values_2, offsets_2, W_2, values_3, offsets_3, W_3, zrows)


# speedup vs baseline: 321.7782x; 1.0045x over previous
import jax
import jax.numpy as jnp
from jax import lax
from jax.experimental import pallas as pl
from jax.experimental.pallas import tpu as pltpu
from jax.experimental.pallas import tpu_sc as plsc

B = 16384
V = 100000
D = 64
T = 4
TOT = B * 20

NC = 2
NS = 16
NW = NC * NS
BPW = B // NW
C = 128
NBUF = 4

OFF_SLICE = 1040
ACC_ROWS = BPW + 1
REGION = NS * ACC_ROWS


def _pool_body(vp0, op0, w0, vp1, op1, w1, vp2, op2, w2, vp3, op3, w3, zrows,
               out, off2, vals4, seg4, rows4, acc_sh,
               sem_v, sem_g, sem_s, sem_f, sem_r, sem_o):
  cid = lax.axis_index("c")
  sid = lax.axis_index("s")
  wid = sid * NC + cid
  bag0 = wid * BPW
  lane = lax.iota(jnp.int32, 16)

  tables = ((vp0, op0, w0), (vp1, op1, w1), (vp2, op2, w2), (vp3, op3, w3))

  pltpu.sync_copy(zrows, acc_sh.at[pl.ds(sid * ACC_ROWS, ACC_ROWS)])
  pltpu.sync_copy(tables[0][1].at[pl.ds(bag0, BPW + 1)],
                  off2.at[pl.ds(0, BPW + 1)])

  for t, (vp, op, w) in enumerate(tables):
    offb = (t % 2) * OFF_SLICE
    accb = (t % 2) * REGION
    nxtb = (1 - t % 2) * REGION
    acc0 = accb + sid * ACC_ROWS
    if t < T - 1:
      if t >= 1:
        pltpu.make_async_copy(
            acc_sh.at[pl.ds(0, BPW)],
            out.at[pl.ds(bag0, BPW), pl.ds(0, D)], sem_o).wait()
      pltpu.async_copy(
          zrows, acc_sh.at[pl.ds(nxtb + sid * ACC_ROWS, ACC_ROWS)], sem_r)
      pltpu.async_copy(
          tables[t + 1][1].at[pl.ds(bag0, BPW + 1)],
          off2.at[pl.ds((1 - t % 2) * OFF_SLICE, BPW + 1)], sem_f)
    if t >= 1:
      pltpu.make_async_copy(zrows, acc_sh.at[pl.ds(0, ACC_ROWS)],
                            sem_r).wait()
      pltpu.make_async_copy(op.at[pl.ds(bag0, BPW + 1)],
                            off2.at[pl.ds(0, BPW + 1)], sem_f).wait()

    start = off2[pl.ds(offb, 16)][0]
    end = off2[pl.ds(offb + BPW, 16)][0]
    astart = lax.bitwise_and(start, jnp.int32(-8))
    nch = (end - astart + (C - 1)) // C

    def cbase_of(m):
      return pl.multiple_of(jnp.minimum(astart + m * C, TOT - C), 8)

    def vals_issue(m, b):
      pltpu.async_copy(vp.at[pl.ds(cbase_of(m), C)], vals4.at[b], sem_v)

    def vals_wait():
      pltpu.make_async_copy(vp.at[pl.ds(0, C)], vals4.at[0], sem_v).wait()

    def gather_issue(b):
      pltpu.async_copy(w.at[vals4.at[b]], rows4.at[b], sem_g)

    def gather_wait():
      pltpu.make_async_copy(w.at[vals4.at[0]], rows4.at[0], sem_g).wait()

    def scatter_issue(b):
      pltpu.async_copy(rows4.at[b], acc_sh.at[seg4.at[b]], sem_s, add=True)

    def scatter_wait():
      pltpu.make_async_copy(rows4.at[0], acc_sh.at[seg4.at[0]], sem_s).wait()

    def seg_compute(m, b, acc0=acc0, offb=offb):
      base = astart + m * C
      cbase = cbase_of(m)

      def one_vreg(j16, carry):
        pos = cbase + j16 * 16 + lane
        j = jnp.zeros((16,), jnp.int32)
        for sh in (512, 256, 128, 64, 32, 16, 8, 4, 2, 1):
          ok = ((j + sh <= BPW + 1) &
                (plsc.load_gather(off2, [j + (sh - 1 + offb)]) <= pos))
          j = jnp.where(ok, j + sh, j)
        seg = jnp.where((j == 0) | (pos < base), jnp.int32(BPW), j - 1) + acc0
        seg4[b, pl.ds(pl.multiple_of(j16 * 16, 16), 16)] = seg
        return carry

      lax.fori_loop(0, C // 16, one_vreg, 0)

    for m in range(NBUF):
      pl.when(m < nch)(lambda m=m: vals_issue(m, m))
    for m in range(2):
      def _prime(m=m):
        vals_wait()
        gather_issue(m)
      pl.when(m < nch)(_prime)

    def group(g, carry):
      for jj in range(NBUF):
        k = g * NBUF + jj
        b2 = (jj + 2) % NBUF
        pl.when((k >= 2) & (k - 2 < nch))(scatter_wait)

        def _launch_next(k=k, b2=b2):
          vals_wait()
          gather_issue(b2)
        pl.when(k + 2 < nch)(_launch_next)

        def _consume(k=k, jj=jj):
          gather_wait()
          seg_compute(k, jj)
          scatter_issue(jj)
        pl.when(k < nch)(_consume)
        pl.when(k + NBUF < nch)(lambda k=k, jj=jj: vals_issue(k + NBUF, jj))
      return carry

    lax.fori_loop(0, (nch + NBUF - 1) // NBUF + 1, group, 0)
    pltpu.async_copy(acc_sh.at[pl.ds(acc0, BPW)],
                     out.at[pl.ds(bag0, BPW), pl.ds(t * D, D)], sem_o)

  for _ in range(2):
    pltpu.make_async_copy(acc_sh.at[pl.ds(0, BPW)],
                          out.at[pl.ds(bag0, BPW), pl.ds(0, D)], sem_o).wait()


_pooled = pl.kernel(
    _pool_body,
    out_type=jax.ShapeDtypeStruct((B, T * D), jnp.float32),
    mesh=plsc.VectorSubcoreMesh(core_axis_name="c", subcore_axis_name="s"),
    compiler_params=pltpu.CompilerParams(
        needs_layout_passes=False, use_tc_tiling_on_sc=False),
    scratch_types=[
        pltpu.VMEM((2 * OFF_SLICE,), jnp.int32),
        pltpu.VMEM((NBUF, C), jnp.int32),
        pltpu.VMEM((NBUF, C), jnp.int32),
        pltpu.VMEM((NBUF, C, D), jnp.float32),
        pltpu.VMEM_SHARED((2 * REGION, D), jnp.float32),
        pltpu.SemaphoreType.DMA,
        pltpu.SemaphoreType.DMA,
        pltpu.SemaphoreType.DMA,
        pltpu.SemaphoreType.DMA,
        pltpu.SemaphoreType.DMA,
        pltpu.SemaphoreType.DMA,
    ],
)


@jax.jit
def kernel(values_0, offsets_0, values_1, offsets_1, values_2, offsets_2,
           values_3, offsets_3, W_0, W_1, W_2, W_3):
  zrows = jnp.zeros((ACC_ROWS, D), jnp.float32)
  return _pooled(values_0, offsets_0, W_0, values_1, offsets_1, W_1,
                 values_2, offsets_2, W_2, values_3, offsets_3, W_3, zrows)


# speedup vs baseline: 326.6624x; 1.0152x over previous
import jax
import jax.numpy as jnp
from jax import lax
from jax.experimental import pallas as pl
from jax.experimental.pallas import tpu as pltpu
from jax.experimental.pallas import tpu_sc as plsc

B = 16384
V = 100000
D = 64
T = 4
TOT = B * 20

NC = 2
NS = 16
NW = NC * NS
BPW = B // NW
C = 128
NBUF = 6
DEPTH = 3

OFF_SLICE = 1040
ACC_ROWS = BPW + 1
REGION = NS * ACC_ROWS


def _pool_body(vp0, op0, w0, vp1, op1, w1, vp2, op2, w2, vp3, op3, w3, zrows,
               out, off2, vals4, seg4, rows4, acc_sh,
               sem_v, sem_g, sem_s, sem_f, sem_r, sem_o):
  cid = lax.axis_index("c")
  sid = lax.axis_index("s")
  wid = sid * NC + cid
  bag0 = wid * BPW
  lane = lax.iota(jnp.int32, 16)

  tables = ((vp0, op0, w0), (vp1, op1, w1), (vp2, op2, w2), (vp3, op3, w3))

  pltpu.sync_copy(zrows, acc_sh.at[pl.ds(sid * ACC_ROWS, ACC_ROWS)])
  pltpu.sync_copy(tables[0][1].at[pl.ds(bag0, BPW + 1)],
                  off2.at[pl.ds(0, BPW + 1)])

  for t, (vp, op, w) in enumerate(tables):
    offb = (t % 2) * OFF_SLICE
    accb = (t % 2) * REGION
    nxtb = (1 - t % 2) * REGION
    acc0 = accb + sid * ACC_ROWS
    if t < T - 1:
      if t >= 1:
        pltpu.make_async_copy(
            acc_sh.at[pl.ds(0, BPW)],
            out.at[pl.ds(bag0, BPW), pl.ds(0, D)], sem_o).wait()
      pltpu.async_copy(
          zrows, acc_sh.at[pl.ds(nxtb + sid * ACC_ROWS, ACC_ROWS)], sem_r)
      pltpu.async_copy(
          tables[t + 1][1].at[pl.ds(bag0, BPW + 1)],
          off2.at[pl.ds((1 - t % 2) * OFF_SLICE, BPW + 1)], sem_f)
    if t >= 1:
      pltpu.make_async_copy(zrows, acc_sh.at[pl.ds(0, ACC_ROWS)],
                            sem_r).wait()
      pltpu.make_async_copy(op.at[pl.ds(bag0, BPW + 1)],
                            off2.at[pl.ds(0, BPW + 1)], sem_f).wait()

    start = off2[pl.ds(offb, 16)][0]
    end = off2[pl.ds(offb + BPW, 16)][0]
    astart = lax.bitwise_and(start, jnp.int32(-8))
    nch = (end - astart + (C - 1)) // C

    def cbase_of(m):
      return pl.multiple_of(jnp.minimum(astart + m * C, TOT - C), 8)

    def vals_issue(m, b):
      pltpu.async_copy(vp.at[pl.ds(cbase_of(m), C)], vals4.at[b], sem_v)

    def vals_wait():
      pltpu.make_async_copy(vp.at[pl.ds(0, C)], vals4.at[0], sem_v).wait()

    def gather_issue(b):
      pltpu.async_copy(w.at[vals4.at[b]], rows4.at[b], sem_g)

    def gather_wait():
      pltpu.make_async_copy(w.at[vals4.at[0]], rows4.at[0], sem_g).wait()

    def scatter_issue(b):
      pltpu.async_copy(rows4.at[b], acc_sh.at[seg4.at[b]], sem_s, add=True)

    def scatter_wait():
      pltpu.make_async_copy(rows4.at[0], acc_sh.at[seg4.at[0]], sem_s).wait()

    def seg_compute(m, b, acc0=acc0, offb=offb):
      base = astart + m * C
      cbase = cbase_of(m)

      def one_vreg(j16, carry):
        pos = cbase + j16 * 16 + lane
        j = jnp.zeros((16,), jnp.int32)
        for sh in (512, 256, 128, 64, 32, 16, 8, 4, 2, 1):
          ok = ((j + sh <= BPW + 1) &
                (plsc.load_gather(off2, [j + (sh - 1 + offb)]) <= pos))
          j = jnp.where(ok, j + sh, j)
        seg = jnp.where((j == 0) | (pos < base), jnp.int32(BPW), j - 1) + acc0
        seg4[b, pl.ds(pl.multiple_of(j16 * 16, 16), 16)] = seg
        return carry

      lax.fori_loop(0, C // 16, one_vreg, 0)

    for m in range(NBUF):
      pl.when(m < nch)(lambda m=m: vals_issue(m, m))
    for m in range(DEPTH):
      def _prime(m=m):
        vals_wait()
        gather_issue(m)
      pl.when(m < nch)(_prime)

    def group(g, carry):
      for jj in range(NBUF):
        k = g * NBUF + jj
        b2 = (jj + DEPTH) % NBUF
        pl.when((k >= DEPTH) & (k - DEPTH < nch))(scatter_wait)

        def _launch_next(k=k, b2=b2):
          vals_wait()
          gather_issue(b2)
        pl.when(k + DEPTH < nch)(_launch_next)

        def _consume(k=k, jj=jj):
          gather_wait()
          seg_compute(k, jj)
          scatter_issue(jj)
        pl.when(k < nch)(_consume)
        pl.when(k + NBUF < nch)(lambda k=k, jj=jj: vals_issue(k + NBUF, jj))
      return carry

    lax.fori_loop(0, (nch + NBUF - 1) // NBUF + 1, group, 0)
    pltpu.async_copy(acc_sh.at[pl.ds(acc0, BPW)],
                     out.at[pl.ds(bag0, BPW), pl.ds(t * D, D)], sem_o)

  for _ in range(2):
    pltpu.make_async_copy(acc_sh.at[pl.ds(0, BPW)],
                          out.at[pl.ds(bag0, BPW), pl.ds(0, D)], sem_o).wait()


_pooled = pl.kernel(
    _pool_body,
    out_type=jax.ShapeDtypeStruct((B, T * D), jnp.float32),
    mesh=plsc.VectorSubcoreMesh(core_axis_name="c", subcore_axis_name="s"),
    compiler_params=pltpu.CompilerParams(
        needs_layout_passes=False, use_tc_tiling_on_sc=False),
    scratch_types=[
        pltpu.VMEM((2 * OFF_SLICE,), jnp.int32),
        pltpu.VMEM((NBUF, C), jnp.int32),
        pltpu.VMEM((NBUF, C), jnp.int32),
        pltpu.VMEM((NBUF, C, D), jnp.float32),
        pltpu.VMEM_SHARED((2 * REGION, D), jnp.float32),
        pltpu.SemaphoreType.DMA,
        pltpu.SemaphoreType.DMA,
        pltpu.SemaphoreType.DMA,
        pltpu.SemaphoreType.DMA,
        pltpu.SemaphoreType.DMA,
        pltpu.SemaphoreType.DMA,
    ],
)


@jax.jit
def kernel(values_0, offsets_0, values_1, offsets_1, values_2, offsets_2,
           values_3, offsets_3, W_0, W_1, W_2, W_3):
  zrows = jnp.zeros((ACC_ROWS, D), jnp.float32)
  return _pooled(values_0, offsets_0, W_0, values_1, offsets_1, W_1,
                 values_2, offsets_2, W_2, values_3, offsets_3, W_3, zrows)


# speedup vs baseline: 331.2446x; 1.0140x over previous
import jax
import jax.numpy as jnp
from jax import lax
from jax.experimental import pallas as pl
from jax.experimental.pallas import tpu as pltpu
from jax.experimental.pallas import tpu_sc as plsc

B = 16384
V = 100000
D = 64
T = 4
TOT = B * 20

NC = 2
NS = 16
NW = NC * NS
BPW = B // NW
C = 128
NBUF = 6
DEPTH = 3

OFF_SLICE = 1040
ACC_ROWS = BPW + 1
REGION = NS * ACC_ROWS


def _pool_body(vp0, op0, w0, vp1, op1, w1, vp2, op2, w2, vp3, op3, w3, zrows,
               out01, out23, off2, vals4, seg4, rows4, acc_sh,
               sem_v, sem_g, sem_s, sem_f, sem_r, sem_o):
  cid = lax.axis_index("c")
  sid = lax.axis_index("s")
  wid = sid * NC + cid
  bag0 = wid * BPW
  lane = lax.iota(jnp.int32, 16)

  tables = ((vp0, op0, w0), (vp1, op1, w1), (vp2, op2, w2), (vp3, op3, w3))

  pltpu.sync_copy(zrows, acc_sh.at[pl.ds(sid * ACC_ROWS, ACC_ROWS)])
  pltpu.sync_copy(tables[0][1].at[pl.ds(bag0, BPW + 1)],
                  off2.at[pl.ds(0, BPW + 1)])

  for t, (vp, op, w) in enumerate(tables):
    offb = (t % 2) * OFF_SLICE
    accb = (t % 2) * REGION
    nxtb = (1 - t % 2) * REGION
    acc0 = accb + sid * ACC_ROWS
    if t < T - 1:
      if t >= 1:
        pltpu.make_async_copy(
            acc_sh.at[pl.ds(0, BPW)],
            out01.at[pl.ds(bag0, BPW), pl.ds(0, D)], sem_o).wait()
      pltpu.async_copy(
          zrows, acc_sh.at[pl.ds(nxtb + sid * ACC_ROWS, ACC_ROWS)], sem_r)
      pltpu.async_copy(
          tables[t + 1][1].at[pl.ds(bag0, BPW + 1)],
          off2.at[pl.ds((1 - t % 2) * OFF_SLICE, BPW + 1)], sem_f)
    if t >= 1:
      pltpu.make_async_copy(zrows, acc_sh.at[pl.ds(0, ACC_ROWS)],
                            sem_r).wait()
      pltpu.make_async_copy(op.at[pl.ds(bag0, BPW + 1)],
                            off2.at[pl.ds(0, BPW + 1)], sem_f).wait()

    start = off2[pl.ds(offb, 16)][0]
    end = off2[pl.ds(offb + BPW, 16)][0]
    astart = lax.bitwise_and(start, jnp.int32(-8))
    nch = (end - astart + (C - 1)) // C

    def cbase_of(m):
      return pl.multiple_of(jnp.minimum(astart + m * C, TOT - C), 8)

    def vals_issue(m, b):
      pltpu.async_copy(vp.at[pl.ds(cbase_of(m), C)], vals4.at[b], sem_v)

    def vals_wait():
      pltpu.make_async_copy(vp.at[pl.ds(0, C)], vals4.at[0], sem_v).wait()

    def gather_issue(b):
      pltpu.async_copy(w.at[vals4.at[b]], rows4.at[b], sem_g)

    def gather_wait():
      pltpu.make_async_copy(w.at[vals4.at[0]], rows4.at[0], sem_g).wait()

    def scatter_issue(b):
      pltpu.async_copy(rows4.at[b], acc_sh.at[seg4.at[b]], sem_s, add=True)

    def scatter_wait():
      pltpu.make_async_copy(rows4.at[0], acc_sh.at[seg4.at[0]], sem_s).wait()

    def seg_compute(m, b, acc0=acc0, offb=offb):
      base = astart + m * C
      cbase = cbase_of(m)

      def one_vreg(j16, carry):
        pos = cbase + j16 * 16 + lane
        j = jnp.zeros((16,), jnp.int32)
        for sh in (512, 256, 128, 64, 32, 16, 8, 4, 2, 1):
          ok = ((j + sh <= BPW + 1) &
                (plsc.load_gather(off2, [j + (sh - 1 + offb)]) <= pos))
          j = jnp.where(ok, j + sh, j)
        seg = jnp.where((j == 0) | (pos < base), jnp.int32(BPW), j - 1) + acc0
        seg4[b, pl.ds(pl.multiple_of(j16 * 16, 16), 16)] = seg
        return carry

      lax.fori_loop(0, C // 16, one_vreg, 0)

    for m in range(NBUF):
      pl.when(m < nch)(lambda m=m: vals_issue(m, m))
    for m in range(DEPTH):
      def _prime(m=m):
        vals_wait()
        gather_issue(m)
      pl.when(m < nch)(_prime)

    def group(g, carry):
      for jj in range(NBUF):
        k = g * NBUF + jj
        b2 = (jj + DEPTH) % NBUF
        pl.when((k >= DEPTH) & (k - DEPTH < nch))(scatter_wait)

        def _launch_next(k=k, b2=b2):
          vals_wait()
          gather_issue(b2)
        pl.when(k + DEPTH < nch)(_launch_next)

        def _consume(k=k, jj=jj):
          gather_wait()
          seg_compute(k, jj)
          scatter_issue(jj)
        pl.when(k < nch)(_consume)
        pl.when(k + NBUF < nch)(lambda k=k, jj=jj: vals_issue(k + NBUF, jj))
      return carry

    lax.fori_loop(0, (nch + NBUF - 1) // NBUF + 1, group, 0)
    outh = out01 if t < 2 else out23
    pltpu.async_copy(acc_sh.at[pl.ds(acc0, BPW)],
                     outh.at[pl.ds(bag0, BPW), pl.ds((t % 2) * D, D)], sem_o)

  for _ in range(2):
    pltpu.make_async_copy(acc_sh.at[pl.ds(0, BPW)],
                          out01.at[pl.ds(bag0, BPW), pl.ds(0, D)], sem_o).wait()


_pooled = pl.kernel(
    _pool_body,
    out_type=(jax.ShapeDtypeStruct((B, 2 * D), jnp.float32),
              jax.ShapeDtypeStruct((B, 2 * D), jnp.float32)),
    mesh=plsc.VectorSubcoreMesh(core_axis_name="c", subcore_axis_name="s"),
    compiler_params=pltpu.CompilerParams(
        needs_layout_passes=False, use_tc_tiling_on_sc=False),
    scratch_types=[
        pltpu.VMEM((2 * OFF_SLICE,), jnp.int32),
        pltpu.VMEM((NBUF, C), jnp.int32),
        pltpu.VMEM((NBUF, C), jnp.int32),
        pltpu.VMEM((NBUF, C, D), jnp.float32),
        pltpu.VMEM_SHARED((2 * REGION, D), jnp.float32),
        pltpu.SemaphoreType.DMA,
        pltpu.SemaphoreType.DMA,
        pltpu.SemaphoreType.DMA,
        pltpu.SemaphoreType.DMA,
        pltpu.SemaphoreType.DMA,
        pltpu.SemaphoreType.DMA,
    ],
)


@jax.jit
def kernel(values_0, offsets_0, values_1, offsets_1, values_2, offsets_2,
           values_3, offsets_3, W_0, W_1, W_2, W_3):
  zrows = jnp.zeros((ACC_ROWS, D), jnp.float32)
  out01, out23 = _pooled(values_0, offsets_0, W_0, values_1, offsets_1, W_1,
                         values_2, offsets_2, W_2, values_3, offsets_3, W_3,
                         zrows)
  return jnp.concatenate([out01, out23], axis=1)
